# Initial kernel scaffold; baseline (speedup 1.0000x reference)
import jax
import jax.numpy as jnp
from jax.experimental import pallas as pl


def kernel(x, table):
    raise NotImplementedError("write your pallas kernel here")


# speedup vs baseline: 1.9836x; 1.9836x over previous
import functools
import jax
import jax.numpy as jnp
from jax import lax
from jax.experimental import pallas as pl
from jax.experimental.pallas import tpu as pltpu
from jax.experimental.pallas import tpu_sc as plsc

N_SEQ = 8192
D_EMB = 1024
BATCH = 4
SEQ = 4096
ROWS = BATCH * SEQ

NC = 2
NS = 16
L = 16
NW = NC * NS
R_PER_W = ROWS // NW
CHUNK = 32
NCHUNK = R_PER_W // CHUNK
NBUF = 2


def _pos_emb_kernel(x_hbm, table_hbm, out_hbm, x_v, idx_v, rows_v, sems):
    wid = lax.axis_index("s") * NC + lax.axis_index("c")
    base = wid * R_PER_W
    j0 = (wid % (SEQ // R_PER_W)) * R_PER_W

    pltpu.sync_copy(x_hbm.at[pl.ds(base, R_PER_W)], x_v)
    for i in range(R_PER_W // L):
        xv = x_v[pl.ds(i * L, L)]
        pos = lax.iota(jnp.int32, L) + (j0 + i * L + 1)
        idx_v[pl.ds(i * L, L)] = jnp.where(xv == 0, 0, pos)

    def issue(c, b):
        pltpu.async_copy(
            table_hbm.at[idx_v.at[pl.ds(c * CHUNK, CHUNK)]], rows_v.at[b],
            sems.at[b])

    issue(0, 0)
    for c in range(NCHUNK):
        if c + 1 < NCHUNK:
            issue(c + 1, (c + 1) % NBUF)
        b = c % NBUF
        pltpu.make_async_copy(
            table_hbm.at[idx_v.at[pl.ds(c * CHUNK, CHUNK)]], rows_v.at[b],
            sems.at[b]).wait()
        pltpu.sync_copy(rows_v.at[b], out_hbm.at[pl.ds(base + c * CHUNK, CHUNK)])


@jax.jit
def kernel(x, table):
    x_flat = x.reshape(ROWS).astype(jnp.int32)
    mesh = plsc.VectorSubcoreMesh(core_axis_name="c", subcore_axis_name="s",
                                  num_cores=NC)
    out = pl.kernel(
        _pos_emb_kernel,
        out_type=jax.ShapeDtypeStruct((ROWS, D_EMB), jnp.float32),
        mesh=mesh,
        scratch_types=[
            pltpu.VMEM((R_PER_W,), jnp.int32),
            pltpu.VMEM((R_PER_W,), jnp.int32),
            pltpu.VMEM((NBUF, CHUNK, D_EMB), jnp.float32),
            pltpu.SemaphoreType.DMA((NBUF,)),
        ],
    )(x_flat, table)
    return out.reshape(BATCH, SEQ, D_EMB)


# speedup vs baseline: 2.6404x; 1.3311x over previous
import jax
import jax.numpy as jnp
from jax import lax
from jax.experimental import pallas as pl
from jax.experimental.pallas import tpu as pltpu
from jax.experimental.pallas import tpu_sc as plsc

N_SEQ = 8192
D_EMB = 1024
BATCH = 4
SEQ = 4096
ROWS = BATCH * SEQ

NC = 2
NS = 16
L = 16
NW = NC * NS
J_PER_W = SEQ // NW
CHUNK = 16
NCHUNK = J_PER_W // CHUNK
NBUF = 2
CHUNK_BYTES = CHUNK * D_EMB * 4


def _pos_emb_kernel(x_hbm, table_hbm, out_hbm,
                    x_v, idx_v, pos_v, nz_v, stage_v, fix_v, sem_g, sem_o):
    wid = lax.axis_index("s") * NC + lax.axis_index("c")
    j0 = wid * J_PER_W

    for b in range(BATCH):
        pltpu.sync_copy(x_hbm.at[pl.ds(b * SEQ + j0, J_PER_W)],
                        x_v.at[pl.ds(b * J_PER_W, J_PER_W)])

    for i in range(J_PER_W // L):
        pos = lax.iota(jnp.int32, L) + (j0 + i * L + 1)
        pos_v[pl.ds(i * L, L)] = pos
        for b in range(BATCH):
            xv = x_v[pl.ds(b * J_PER_W + i * L, L)]
            idx_v[pl.ds(b * J_PER_W + i * L, L)] = jnp.where(xv == 0, 0, pos)

    def issue_gather(c):
        pltpu.async_copy(
            table_hbm.at[pos_v.at[pl.ds(c * CHUNK, CHUNK)]],
            stage_v.at[c % NBUF], sem_g.at[c % NBUF])

    def drain_out(p):
        for _ in range(BATCH):
            pltpu.make_async_copy(
                stage_v.at[p], out_hbm.at[pl.ds(0, CHUNK)], sem_o.at[p]).wait()

    issue_gather(0)
    for c in range(NCHUNK):
        p = c % NBUF
        if c >= 1:
            drain_out((c - 1) % NBUF)
        if c + 1 < NCHUNK:
            issue_gather(c + 1)
        pltpu.make_async_copy(
            table_hbm.at[pos_v.at[pl.ds(c * CHUNK, CHUNK)]],
            stage_v.at[p], sem_g.at[p]).wait()

        for b in range(BATCH):
            xv0 = x_v[pl.ds(b * J_PER_W + c * CHUNK, L)]
            v = jnp.where(xv0 == 0, 1, 0).astype(jnp.int32)
            lane = lax.iota(jnp.int32, L)
            dnums = lax.GatherDimensionNumbers(
                offset_dims=(), collapsed_slice_dims=(0,),
                start_index_map=(0,))
            for sh in (8, 4, 2, 1):
                rot = lax.gather(
                    v, ((lane + sh) & (L - 1))[:, None], dnums,
                    slice_sizes=(1,),
                    mode=lax.GatherScatterMode.PROMISE_IN_BOUNDS)
                v = v + rot
            nz = v[0]
            dst = out_hbm.at[pl.ds(b * SEQ + j0 + c * CHUNK, CHUNK)]

            @pl.when(nz == 0)
            def _fast():
                pltpu.async_copy(stage_v.at[p], dst, sem_o.at[p])

            @pl.when(nz != 0)
            def _slow():
                pltpu.sync_copy(
                    table_hbm.at[idx_v.at[pl.ds(b * J_PER_W + c * CHUNK,
                                                CHUNK)]],
                    fix_v.at[b])
                pltpu.async_copy(fix_v.at[b], dst, sem_o.at[p])

    drain_out((NCHUNK - 1) % NBUF)


@jax.jit
def kernel(x, table):
    x_flat = x.reshape(ROWS).astype(jnp.int32)
    mesh = plsc.VectorSubcoreMesh(core_axis_name="c", subcore_axis_name="s",
                                  num_cores=NC)
    out = pl.kernel(
        _pos_emb_kernel,
        out_type=jax.ShapeDtypeStruct((ROWS, D_EMB), jnp.float32),
        mesh=mesh,
        scratch_types=[
            pltpu.VMEM((BATCH * J_PER_W,), jnp.int32),
            pltpu.VMEM((BATCH * J_PER_W,), jnp.int32),
            pltpu.VMEM((J_PER_W,), jnp.int32),
            pltpu.VMEM((L,), jnp.int32),
            pltpu.VMEM((NBUF, CHUNK, D_EMB), jnp.float32),
            pltpu.VMEM((BATCH, CHUNK, D_EMB), jnp.float32),
            pltpu.SemaphoreType.DMA((NBUF,)),
            pltpu.SemaphoreType.DMA((NBUF,)),
        ],
    )(x_flat, table)
    return out.reshape(BATCH, SEQ, D_EMB)
